# baseline (speedup 1.0000x reference)
import functools

import jax
import jax.numpy as jnp
from jax import lax
from jax.experimental import pallas as pl
from jax.experimental.pallas import tpu as pltpu
from jax.experimental.pallas import tpu_sc as plsc

N = 10000
E = 320000
IN = 128
OUT = 128
H = 8
D = 16

NC = 2
NS = 16
NW = NC * NS

NP = 10240
ROWS_T = NP // NS

CH = 128
CPW = 80
EPW = CH * CPW
EP = NW * EPW

NEG = -1e30


def _k1_body(x_ref, wt_ref, b_ref, ms_ref, md_ref, h_ref, as_ref, ad_ref):
    i = pl.program_id(0)
    h = jnp.dot(x_ref[...], wt_ref[...], preferred_element_type=jnp.float32)
    h = h + b_ref[...]
    h_ref[...] = h
    rows = i * 512 + lax.broadcasted_iota(jnp.int32, (512, 1), 0)
    valid = rows < N
    a_s = jnp.dot(h, ms_ref[...], preferred_element_type=jnp.float32)
    a_d = jnp.dot(h, md_ref[...], preferred_element_type=jnp.float32)
    as_ref[...] = jnp.where(valid, a_s, NEG)
    ad_ref[...] = jnp.where(valid, a_d, NEG)


def _k1(x_pad, wt, b2, ms, md):
    return pl.pallas_call(
        _k1_body,
        grid=(NP // 512,),
        in_specs=[
            pl.BlockSpec((512, IN), lambda i: (i, 0)),
            pl.BlockSpec((IN, OUT), lambda i: (0, 0)),
            pl.BlockSpec((1, OUT), lambda i: (0, 0)),
            pl.BlockSpec((OUT, 16), lambda i: (0, 0)),
            pl.BlockSpec((OUT, 16), lambda i: (0, 0)),
        ],
        out_specs=[
            pl.BlockSpec((512, OUT), lambda i: (i, 0)),
            pl.BlockSpec((512, 16), lambda i: (i, 0)),
            pl.BlockSpec((512, 16), lambda i: (i, 0)),
        ],
        out_shape=[
            jax.ShapeDtypeStruct((NP, OUT), jnp.float32),
            jax.ShapeDtypeStruct((NP, 16), jnp.float32),
            jax.ShapeDtypeStruct((NP, 16), jnp.float32),
        ],
    )(x_pad, wt, b2, ms, md)


def _k2_body(cidxp, as_hbm, ad_hbm,
             ex_hbm, den_hbm,
             cidx, s_ch, d_ch, ex_ch, zbuf, den_sh,
             sem0, sem1, sem2, sem3, sem4, sem5):
    cid = lax.axis_index("c")
    sid = lax.axis_index("s")
    wid = cid * NS + sid

    def _z(i, c):
        zbuf[i, :] = jnp.zeros((16,), jnp.float32)
        return c
    lax.fori_loop(0, 40, _z, 0)

    def _z2(t, c):
        pltpu.sync_copy(zbuf, den_sh.at[pl.ds(sid * ROWS_T + t * 40, 40)])
        return c
    lax.fori_loop(0, ROWS_T // 40, _z2, 0)
    plsc.subcore_barrier()

    gsem = (sem0, sem1)
    ssem = (sem2, sem3)
    esem = (sem4, sem5)

    def _fetch(j, p):
        row = wid * CPW + j
        pltpu.sync_copy(cidxp.at[pl.ds(row, 1)], cidx.at[pl.ds(p, 1)])
        pltpu.async_copy(as_hbm.at[cidx.at[p, 0]], s_ch.at[p], gsem[p])
        pltpu.async_copy(ad_hbm.at[cidx.at[p, 1]], d_ch.at[p], gsem[p])

    def _wait_fetch(p):
        pltpu.make_async_copy(as_hbm.at[cidx.at[p, 0]], s_ch.at[p],
                              gsem[p]).wait()
        pltpu.make_async_copy(ad_hbm.at[cidx.at[p, 1]], d_ch.at[p],
                              gsem[p]).wait()

    def _compute_scatter(j, p):
        row = wid * CPW + j

        def _edge(e, c2):
            v = s_ch[p, e, :] + d_ch[p, e, :]
            v = jnp.where(v > 0, v, v * jnp.float32(0.2))
            ex_ch[p, e, :] = jnp.exp(v)
            return c2
        lax.fori_loop(0, CH, _edge, 0)
        pltpu.async_copy(ex_ch.at[p], den_sh.at[cidx.at[p, 1]], ssem[p],
                         add=True)
        pltpu.async_copy(ex_ch.at[p], ex_hbm.at[pl.ds(row * CH, CH)],
                         esem[p])

    def _wait_scatter(p):
        pltpu.make_async_copy(ex_ch.at[p], den_sh.at[cidx.at[p, 1]],
                              ssem[p]).wait()
        pltpu.make_async_copy(ex_ch.at[p], ex_hbm.at[pl.ds(0, CH)],
                              esem[p]).wait()

    _fetch(0, 0)
    _fetch(1, 1)
    _wait_fetch(0)
    _compute_scatter(0, 0)

    def _step(j, p):
        q = 1 - p
        _wait_fetch(p)
        _wait_scatter(q)
        _fetch(j + 1, q)
        _compute_scatter(j, p)

    def _steady(jj, c):
        j = 2 * jj + 1
        _step(j, 1)
        _step(j + 1, 0)
        return c
    lax.fori_loop(0, (CPW - 2) // 2, _steady, 0)

    p = (CPW - 1) % 2
    _wait_fetch(p)
    _wait_scatter(1 - p)
    _compute_scatter(CPW - 1, p)
    _wait_scatter(p)

    plsc.subcore_barrier()
    pltpu.sync_copy(den_sh.at[pl.ds(sid * ROWS_T, ROWS_T)],
                    den_hbm.at[cid, pl.ds(sid * ROWS_T, ROWS_T)])


def _k2(cidxp, as_t, ad_t):
    mesh = plsc.VectorSubcoreMesh(core_axis_name="c", subcore_axis_name="s")
    f = pl.kernel(
        _k2_body,
        compiler_params=pltpu.CompilerParams(use_tc_tiling_on_sc=False),
        out_type=[
            jax.ShapeDtypeStruct((EP, 16), jnp.float32),
            jax.ShapeDtypeStruct((NC, NP, 16), jnp.float32),
        ],
        mesh=mesh,
        scratch_types=[
            pltpu.VMEM((2, 2, CH), jnp.int32),
            pltpu.VMEM((2, CH, 16), jnp.float32),
            pltpu.VMEM((2, CH, 16), jnp.float32),
            pltpu.VMEM((2, CH, 16), jnp.float32),
            pltpu.VMEM((40, 16), jnp.float32),
            pltpu.VMEM_SHARED((NP, 16), jnp.float32),
            pltpu.SemaphoreType.DMA,
            pltpu.SemaphoreType.DMA,
            pltpu.SemaphoreType.DMA,
            pltpu.SemaphoreType.DMA,
            pltpu.SemaphoreType.DMA,
            pltpu.SemaphoreType.DMA,
        ],
    )
    return f(cidxp, as_t, ad_t)


def _k2b_body(den_ref, r_ref):
    r_ref[...] = 1.0 / (den_ref[0] + den_ref[1] + jnp.float32(1e-16))


def _k2b(den):
    return pl.pallas_call(
        _k2b_body,
        out_shape=jax.ShapeDtypeStruct((NP, 16), jnp.float32),
    )(den)


def _k3_body(cidxp, ex_hbm, r_hbm, h_hbm,
             agg_hbm,
             cidx, r_ch, ex_ch, hs, zbuf,
             agg_sh, gsem0, gsem1, ssem0, ssem1, isem0, isem1, isem2):
    cid = lax.axis_index("c")
    sid = lax.axis_index("s")
    wid = cid * NS + sid
    gsem = (gsem0, gsem1)
    ssem = (ssem0, ssem1)
    isem = (isem0, isem1, isem2)

    def _z(i, c):
        r = i // 8
        cc = i % 8
        zbuf[r, pl.ds(cc * 16, 16)] = jnp.zeros((16,), jnp.float32)
        return c
    lax.fori_loop(0, 8 * 8, _z, 0)

    def _z2(t, c):
        pltpu.sync_copy(zbuf, agg_sh.at[pl.ds(sid * ROWS_T + t * 8, 8)])
        return c
    lax.fori_loop(0, ROWS_T // 8, _z2, 0)
    plsc.subcore_barrier()

    def _issue_idx(j, r):
        row = wid * CPW + jnp.minimum(j, CPW - 1)
        pltpu.async_copy(cidxp.at[pl.ds(row, 1)], cidx.at[pl.ds(r, 1)],
                         isem[r])

    def _wait_idx(r):
        pltpu.make_async_copy(cidxp.at[pl.ds(0, 1)], cidx.at[pl.ds(r, 1)],
                              isem[r]).wait()

    def _gathers(j, p, r):
        row = wid * CPW + j
        pltpu.async_copy(r_hbm.at[cidx.at[r, 1]], r_ch.at[p], gsem[p])
        pltpu.async_copy(h_hbm.at[cidx.at[r, 0]], hs.at[p], gsem[p])
        pltpu.async_copy(ex_hbm.at[pl.ds(row * CH, CH)], ex_ch.at[p], gsem[p])

    def _wait_fetch(p, r):
        pltpu.make_async_copy(r_hbm.at[cidx.at[r, 1]], r_ch.at[p],
                              gsem[p]).wait()
        pltpu.make_async_copy(h_hbm.at[cidx.at[r, 0]], hs.at[p],
                              gsem[p]).wait()
        pltpu.make_async_copy(ex_hbm.at[pl.ds(0, CH)], ex_ch.at[p],
                              gsem[p]).wait()

    def _compute_scatter(p, r):
        def _e(e, c2):
            wrow = ex_ch[p, e, :] * r_ch[p, e, :]
            for hh in range(H):
                hs[p, e, pl.ds(hh * D, D)] = (hs[p, e, pl.ds(hh * D, D)]
                                              * wrow[hh])
            return c2
        lax.fori_loop(0, CH, _e, 0, unroll=4)
        pltpu.async_copy(hs.at[p], agg_sh.at[cidx.at[r, 1]], ssem[p],
                         add=True)

    def _wait_scatter(p, r):
        pltpu.make_async_copy(hs.at[p], agg_sh.at[cidx.at[r, 1]],
                              ssem[p]).wait()

    pltpu.sync_copy(cidxp.at[pl.ds(wid * CPW, 1)], cidx.at[pl.ds(0, 1)])
    pltpu.sync_copy(cidxp.at[pl.ds(wid * CPW + 1, 1)], cidx.at[pl.ds(1, 1)])
    _issue_idx(2, 2)
    _gathers(0, 0, 0)
    _gathers(1, 1, 1)
    _wait_fetch(0, 0)
    _compute_scatter(0, 0)

    def _stepx(j, p, rj, r1, r2):
        q = 1 - p
        _wait_fetch(p, rj)
        _wait_scatter(q, r2)
        _wait_idx(r1)
        _gathers(j + 1, q, r1)
        _issue_idx(j + 2, r2)
        _compute_scatter(p, rj)

    def _steady(jj, c):
        for m in range(6):
            j = 6 * jj + 1 + m
            _stepx(j, (1 + m) % 2, (1 + m) % 3, (2 + m) % 3, (m) % 3)
        return c
    lax.fori_loop(0, (CPW - 2) // 6, _steady, 0)

    p = (CPW - 1) % 2
    rj = (CPW - 1) % 3
    _wait_fetch(p, rj)
    _wait_scatter(1 - p, (CPW - 2) % 3)
    _wait_idx(CPW % 3)
    _compute_scatter(p, rj)
    _wait_scatter(p, rj)

    plsc.subcore_barrier()
    pltpu.sync_copy(agg_sh.at[pl.ds(sid * ROWS_T, ROWS_T)],
                    agg_hbm.at[cid, pl.ds(sid * ROWS_T, ROWS_T)])


def _k3(cidxp, ex, r, h):
    mesh = plsc.VectorSubcoreMesh(core_axis_name="c", subcore_axis_name="s")
    f = pl.kernel(
        _k3_body,
        compiler_params=pltpu.CompilerParams(use_tc_tiling_on_sc=False),
        out_type=jax.ShapeDtypeStruct((NC, NP, OUT), jnp.float32),
        mesh=mesh,
        scratch_types=[
            pltpu.VMEM((3, 2, CH), jnp.int32),
            pltpu.VMEM((2, CH, 16), jnp.float32),
            pltpu.VMEM((2, CH, 16), jnp.float32),
            pltpu.VMEM((2, CH, OUT), jnp.float32),
            pltpu.VMEM((8, 128), jnp.float32),
            pltpu.VMEM_SHARED((NP, OUT), jnp.float32),
            pltpu.SemaphoreType.DMA,
            pltpu.SemaphoreType.DMA,
            pltpu.SemaphoreType.DMA,
            pltpu.SemaphoreType.DMA,
            pltpu.SemaphoreType.DMA,
            pltpu.SemaphoreType.DMA,
            pltpu.SemaphoreType.DMA,
        ],
    )
    return f(cidxp, ex, r, h)


def _k4_body(agg_ref, g_ref, b_ref, o_ref):
    out = jax.nn.relu(agg_ref[0, :N, :] + agg_ref[1, :N, :])
    mean = jnp.mean(out, axis=0, keepdims=True)
    var = jnp.mean((out - mean) ** 2, axis=0, keepdims=True)
    o_ref[...] = (out - mean) / jnp.sqrt(var + 1e-5) * g_ref[...] + b_ref[...]


def _k4(agg, gamma, beta):
    return pl.pallas_call(
        _k4_body,
        out_shape=jax.ShapeDtypeStruct((N, OUT), jnp.float32),
    )(agg, gamma.reshape(1, OUT), beta.reshape(1, OUT))


def kernel(x, edge_index, W_proj, b_proj, att_src, att_dst, k_W, k_b, q,
           bn_gamma, bn_beta):
    mask = jnp.concatenate([jnp.eye(H, dtype=jnp.float32),
                            jnp.zeros((H, H), jnp.float32)], axis=1)
    ms = (att_src.reshape(H, D)[:, :, None] * mask[:, None, :]).reshape(OUT, 16)
    md = (att_dst.reshape(H, D)[:, :, None] * mask[:, None, :]).reshape(OUT, 16)

    x_pad = jnp.zeros((NP, IN), jnp.float32).at[:N, :].set(x)
    h, as_t, ad_t = _k1(x_pad, W_proj.T, b_proj.reshape(1, OUT), ms, md)

    src = edge_index[0]
    dst = edge_index[1]
    pad_idx = jnp.full((EP - E,), N, jnp.int32)
    srcp = jnp.concatenate([src, pad_idx]).reshape(NW * CPW, CH)
    dstp = jnp.concatenate([dst, pad_idx]).reshape(NW * CPW, CH)
    cidxp = jnp.stack([srcp, dstp], axis=1)

    ex, den = _k2(cidxp, as_t, ad_t)
    r = _k2b(den)
    agg = _k3(cidxp, ex, r, h)
    return _k4(agg, bn_gamma, bn_beta)


# speedup vs baseline: 1.0558x; 1.0558x over previous
import functools

import jax
import jax.numpy as jnp
from jax import lax
from jax.experimental import pallas as pl
from jax.experimental.pallas import tpu as pltpu
from jax.experimental.pallas import tpu_sc as plsc

N = 10000
E = 320000
IN = 128
OUT = 128
H = 8
D = 16

NC = 2
NS = 16
NW = NC * NS

NP = 10240
ROWS_T = NP // NS

CH = 128
CPW = 80
EPW = CH * CPW
EP = NW * EPW
CPW0 = 68
CPW1 = 92

NEG = -1e30


def _k1_body(x_ref, wt_ref, b_ref, ms_ref, md_ref, h_ref, as_ref, ad_ref):
    i = pl.program_id(0)
    h = jnp.dot(x_ref[...], wt_ref[...], preferred_element_type=jnp.float32)
    h = h + b_ref[...]
    h_ref[...] = h
    rows = i * 512 + lax.broadcasted_iota(jnp.int32, (512, 1), 0)
    valid = rows < N
    a_s = jnp.dot(h, ms_ref[...], preferred_element_type=jnp.float32)
    a_d = jnp.dot(h, md_ref[...], preferred_element_type=jnp.float32)
    as_ref[...] = jnp.where(valid, a_s, NEG)
    ad_ref[...] = jnp.where(valid, a_d, NEG)


def _k1(x_pad, wt, b2, ms, md):
    return pl.pallas_call(
        _k1_body,
        grid=(NP // 512,),
        in_specs=[
            pl.BlockSpec((512, IN), lambda i: (i, 0)),
            pl.BlockSpec((IN, OUT), lambda i: (0, 0)),
            pl.BlockSpec((1, OUT), lambda i: (0, 0)),
            pl.BlockSpec((OUT, 16), lambda i: (0, 0)),
            pl.BlockSpec((OUT, 16), lambda i: (0, 0)),
        ],
        out_specs=[
            pl.BlockSpec((512, OUT), lambda i: (i, 0)),
            pl.BlockSpec((512, 16), lambda i: (i, 0)),
            pl.BlockSpec((512, 16), lambda i: (i, 0)),
        ],
        out_shape=[
            jax.ShapeDtypeStruct((NP, OUT), jnp.float32),
            jax.ShapeDtypeStruct((NP, 16), jnp.float32),
            jax.ShapeDtypeStruct((NP, 16), jnp.float32),
        ],
    )(x_pad, wt, b2, ms, md)


def _k2_body(cidxp, as_hbm, ad_hbm,
             ex_hbm, den_hbm,
             cidx, s_ch, d_ch, ex_ch, zbuf, den_sh,
             sem0, sem1, sem2, sem3, sem4, sem5):
    cid = lax.axis_index("c")
    sid = lax.axis_index("s")
    wid = cid * NS + sid

    def _z(i, c):
        zbuf[i, :] = jnp.zeros((16,), jnp.float32)
        return c
    lax.fori_loop(0, 40, _z, 0)

    def _z2(t, c):
        pltpu.sync_copy(zbuf, den_sh.at[pl.ds(sid * ROWS_T + t * 40, 40)])
        return c
    lax.fori_loop(0, ROWS_T // 40, _z2, 0)
    plsc.subcore_barrier()

    gsem = (sem0, sem1)
    ssem = (sem2, sem3)
    esem = (sem4, sem5)

    def _fetch(j, p):
        row = wid * CPW + j
        pltpu.sync_copy(cidxp.at[pl.ds(row, 1)], cidx.at[pl.ds(p, 1)])
        pltpu.async_copy(as_hbm.at[cidx.at[p, 0]], s_ch.at[p], gsem[p])
        pltpu.async_copy(ad_hbm.at[cidx.at[p, 1]], d_ch.at[p], gsem[p])

    def _wait_fetch(p):
        pltpu.make_async_copy(as_hbm.at[cidx.at[p, 0]], s_ch.at[p],
                              gsem[p]).wait()
        pltpu.make_async_copy(ad_hbm.at[cidx.at[p, 1]], d_ch.at[p],
                              gsem[p]).wait()

    def _compute_scatter(j, p):
        row = wid * CPW + j

        def _edge(e, c2):
            v = s_ch[p, e, :] + d_ch[p, e, :]
            v = jnp.where(v > 0, v, v * jnp.float32(0.2))
            ex_ch[p, e, :] = jnp.exp(v)
            return c2
        lax.fori_loop(0, CH, _edge, 0)
        pltpu.async_copy(ex_ch.at[p], den_sh.at[cidx.at[p, 1]], ssem[p],
                         add=True)
        pltpu.async_copy(ex_ch.at[p], ex_hbm.at[pl.ds(row * CH, CH)],
                         esem[p])

    def _wait_scatter(p):
        pltpu.make_async_copy(ex_ch.at[p], den_sh.at[cidx.at[p, 1]],
                              ssem[p]).wait()
        pltpu.make_async_copy(ex_ch.at[p], ex_hbm.at[pl.ds(0, CH)],
                              esem[p]).wait()

    _fetch(0, 0)
    _fetch(1, 1)
    _wait_fetch(0)
    _compute_scatter(0, 0)

    def _step(j, p):
        q = 1 - p
        _wait_fetch(p)
        _wait_scatter(q)
        _fetch(j + 1, q)
        _compute_scatter(j, p)

    def _steady(jj, c):
        j = 2 * jj + 1
        _step(j, 1)
        _step(j + 1, 0)
        return c
    lax.fori_loop(0, (CPW - 2) // 2, _steady, 0)

    p = (CPW - 1) % 2
    _wait_fetch(p)
    _wait_scatter(1 - p)
    _compute_scatter(CPW - 1, p)
    _wait_scatter(p)

    plsc.subcore_barrier()
    pltpu.sync_copy(den_sh.at[pl.ds(sid * ROWS_T, ROWS_T)],
                    den_hbm.at[cid, pl.ds(sid * ROWS_T, ROWS_T)])


def _k2(cidxp, as_t, ad_t):
    mesh = plsc.VectorSubcoreMesh(core_axis_name="c", subcore_axis_name="s")
    f = pl.kernel(
        _k2_body,
        compiler_params=pltpu.CompilerParams(use_tc_tiling_on_sc=False),
        out_type=[
            jax.ShapeDtypeStruct((EP, 16), jnp.float32),
            jax.ShapeDtypeStruct((NC, NP, 16), jnp.float32),
        ],
        mesh=mesh,
        scratch_types=[
            pltpu.VMEM((2, 2, CH), jnp.int32),
            pltpu.VMEM((2, CH, 16), jnp.float32),
            pltpu.VMEM((2, CH, 16), jnp.float32),
            pltpu.VMEM((2, CH, 16), jnp.float32),
            pltpu.VMEM((40, 16), jnp.float32),
            pltpu.VMEM_SHARED((NP, 16), jnp.float32),
            pltpu.SemaphoreType.DMA,
            pltpu.SemaphoreType.DMA,
            pltpu.SemaphoreType.DMA,
            pltpu.SemaphoreType.DMA,
            pltpu.SemaphoreType.DMA,
            pltpu.SemaphoreType.DMA,
        ],
    )
    return f(cidxp, as_t, ad_t)


def _k2b_body(den_ref, r_ref):
    r_ref[...] = 1.0 / (den_ref[0] + den_ref[1] + jnp.float32(1e-16))


def _k2b(den):
    return pl.pallas_call(
        _k2b_body,
        out_shape=jax.ShapeDtypeStruct((NP, 16), jnp.float32),
    )(den)


def _k3_body(cidxp, ex_hbm, r_hbm, h_hbm,
             agg_hbm,
             cidx, r_ch, ex_ch, hs, zbuf,
             agg_sh, gsem0, gsem1, ssem0, ssem1, isem0, isem1, isem2):
    cid = lax.axis_index("c")
    sid = lax.axis_index("s")
    wid = cid * NS + sid
    gsem = (gsem0, gsem1)
    ssem = (ssem0, ssem1)
    isem = (isem0, isem1, isem2)

    def _z(i, c):
        r = i // 8
        cc = i % 8
        zbuf[r, pl.ds(cc * 16, 16)] = jnp.zeros((16,), jnp.float32)
        return c
    lax.fori_loop(0, 8 * 8, _z, 0)

    def _z2(t, c):
        pltpu.sync_copy(zbuf, agg_sh.at[pl.ds(sid * ROWS_T + t * 8, 8)])
        return c
    lax.fori_loop(0, ROWS_T // 8, _z2, 0)
    plsc.subcore_barrier()

    def _pipeline(base, cpw):
        def _issue_idx(j, r):
            row = base + jnp.minimum(j, cpw - 1)
            pltpu.async_copy(cidxp.at[pl.ds(row, 1)], cidx.at[pl.ds(r, 1)],
                             isem[r])

        def _wait_idx(r):
            pltpu.make_async_copy(cidxp.at[pl.ds(0, 1)],
                                  cidx.at[pl.ds(r, 1)], isem[r]).wait()

        def _gathers(j, p, r):
            row = base + j
            pltpu.async_copy(r_hbm.at[cidx.at[r, 1]], r_ch.at[p], gsem[p])
            pltpu.async_copy(h_hbm.at[cidx.at[r, 0]], hs.at[p], gsem[p])
            pltpu.async_copy(ex_hbm.at[pl.ds(row * CH, CH)], ex_ch.at[p],
                             gsem[p])

        def _wait_fetch(p, r):
            pltpu.make_async_copy(r_hbm.at[cidx.at[r, 1]], r_ch.at[p],
                                  gsem[p]).wait()
            pltpu.make_async_copy(h_hbm.at[cidx.at[r, 0]], hs.at[p],
                                  gsem[p]).wait()
            pltpu.make_async_copy(ex_hbm.at[pl.ds(0, CH)], ex_ch.at[p],
                                  gsem[p]).wait()

        def _compute_scatter(p, r):
            def _e(e, c2):
                wrow = ex_ch[p, e, :] * r_ch[p, e, :]
                for hh in range(H):
                    hs[p, e, pl.ds(hh * D, D)] = (hs[p, e, pl.ds(hh * D, D)]
                                                  * wrow[hh])
                return c2
            lax.fori_loop(0, CH, _e, 0)
            pltpu.async_copy(hs.at[p], agg_sh.at[cidx.at[r, 1]], ssem[p],
                             add=True)

        def _wait_scatter(p, r):
            pltpu.make_async_copy(hs.at[p], agg_sh.at[cidx.at[r, 1]],
                                  ssem[p]).wait()

        pltpu.sync_copy(cidxp.at[pl.ds(base, 1)], cidx.at[pl.ds(0, 1)])
        pltpu.sync_copy(cidxp.at[pl.ds(base + 1, 1)], cidx.at[pl.ds(1, 1)])
        _issue_idx(2, 2)
        _gathers(0, 0, 0)
        _gathers(1, 1, 1)
        _wait_fetch(0, 0)
        _compute_scatter(0, 0)

        def _stepx(j, p, rj, r1, r2):
            q = 1 - p
            _wait_fetch(p, rj)
            _wait_scatter(q, r2)
            _wait_idx(r1)
            _gathers(j + 1, q, r1)
            _issue_idx(j + 2, r2)
            _compute_scatter(p, rj)

        def _steady(jj, c):
            for m in range(6):
                j = 6 * jj + 1 + m
                _stepx(j, (1 + m) % 2, (1 + m) % 3, (2 + m) % 3, m % 3)
            return c
        lax.fori_loop(0, (cpw - 2) // 6, _steady, 0)

        p = (cpw - 1) % 2
        rj = (cpw - 1) % 3
        _wait_fetch(p, rj)
        _wait_scatter(1 - p, (cpw - 2) % 3)
        _wait_idx(cpw % 3)
        _compute_scatter(p, rj)
        _wait_scatter(p, rj)

    @pl.when(cid == 0)
    def _core0():
        _pipeline(sid * CPW0, CPW0)

    @pl.when(cid == 1)
    def _core1():
        _pipeline(NS * CPW0 + sid * CPW1, CPW1)

    plsc.subcore_barrier()
    pltpu.sync_copy(agg_sh.at[pl.ds(sid * ROWS_T, ROWS_T)],
                    agg_hbm.at[cid, pl.ds(sid * ROWS_T, ROWS_T)])


def _k3(cidxp, ex, r, h):
    mesh = plsc.VectorSubcoreMesh(core_axis_name="c", subcore_axis_name="s")
    f = pl.kernel(
        _k3_body,
        compiler_params=pltpu.CompilerParams(use_tc_tiling_on_sc=False),
        out_type=jax.ShapeDtypeStruct((NC, NP, OUT), jnp.float32),
        mesh=mesh,
        scratch_types=[
            pltpu.VMEM((3, 2, CH), jnp.int32),
            pltpu.VMEM((2, CH, 16), jnp.float32),
            pltpu.VMEM((2, CH, 16), jnp.float32),
            pltpu.VMEM((2, CH, OUT), jnp.float32),
            pltpu.VMEM((8, 128), jnp.float32),
            pltpu.VMEM_SHARED((NP, OUT), jnp.float32),
            pltpu.SemaphoreType.DMA,
            pltpu.SemaphoreType.DMA,
            pltpu.SemaphoreType.DMA,
            pltpu.SemaphoreType.DMA,
            pltpu.SemaphoreType.DMA,
            pltpu.SemaphoreType.DMA,
            pltpu.SemaphoreType.DMA,
        ],
    )
    return f(cidxp, ex, r, h)


def _k4_body(agg_ref, g_ref, b_ref, o_ref):
    out = jax.nn.relu(agg_ref[0, :N, :] + agg_ref[1, :N, :])
    mean = jnp.mean(out, axis=0, keepdims=True)
    var = jnp.mean((out - mean) ** 2, axis=0, keepdims=True)
    o_ref[...] = (out - mean) / jnp.sqrt(var + 1e-5) * g_ref[...] + b_ref[...]


def _k4(agg, gamma, beta):
    return pl.pallas_call(
        _k4_body,
        out_shape=jax.ShapeDtypeStruct((N, OUT), jnp.float32),
    )(agg, gamma.reshape(1, OUT), beta.reshape(1, OUT))


def kernel(x, edge_index, W_proj, b_proj, att_src, att_dst, k_W, k_b, q,
           bn_gamma, bn_beta):
    mask = jnp.concatenate([jnp.eye(H, dtype=jnp.float32),
                            jnp.zeros((H, H), jnp.float32)], axis=1)
    ms = (att_src.reshape(H, D)[:, :, None] * mask[:, None, :]).reshape(OUT, 16)
    md = (att_dst.reshape(H, D)[:, :, None] * mask[:, None, :]).reshape(OUT, 16)

    x_pad = jnp.zeros((NP, IN), jnp.float32).at[:N, :].set(x)
    h, as_t, ad_t = _k1(x_pad, W_proj.T, b_proj.reshape(1, OUT), ms, md)

    src = edge_index[0]
    dst = edge_index[1]
    pad_idx = jnp.full((EP - E,), N, jnp.int32)
    srcp = jnp.concatenate([src, pad_idx]).reshape(NW * CPW, CH)
    dstp = jnp.concatenate([dst, pad_idx]).reshape(NW * CPW, CH)
    cidxp = jnp.stack([srcp, dstp], axis=1)

    ex, den = _k2(cidxp, as_t, ad_t)
    r = _k2b(den)
    agg = _k3(cidxp, ex, r, h)
    return _k4(agg, bn_gamma, bn_beta)


# speedup vs baseline: 1.1070x; 1.0485x over previous
import functools

import jax
import jax.numpy as jnp
from jax import lax
from jax.experimental import pallas as pl
from jax.experimental.pallas import tpu as pltpu
from jax.experimental.pallas import tpu_sc as plsc

N = 10000
E = 320000
IN = 128
OUT = 128
H = 8
D = 16

NC = 2
NS = 16
NW = NC * NS

NP = 10240
ROWS_T = NP // NS

CH = 128
CPW = 80
EPW = CH * CPW
EP = NW * EPW
CPW0 = 92
CPW1 = 68

NEG = -1e30


def _k1_body(x_ref, wt_ref, b_ref, ms_ref, md_ref, h_ref, as_ref, ad_ref):
    i = pl.program_id(0)
    h = jnp.dot(x_ref[...], wt_ref[...], preferred_element_type=jnp.float32)
    h = h + b_ref[...]
    h_ref[...] = h
    rows = i * 512 + lax.broadcasted_iota(jnp.int32, (512, 1), 0)
    valid = rows < N
    a_s = jnp.dot(h, ms_ref[...], preferred_element_type=jnp.float32)
    a_d = jnp.dot(h, md_ref[...], preferred_element_type=jnp.float32)
    as_ref[...] = jnp.where(valid, a_s, NEG)
    ad_ref[...] = jnp.where(valid, a_d, NEG)


def _k1(x_pad, wt, b2, ms, md):
    return pl.pallas_call(
        _k1_body,
        grid=(NP // 512,),
        in_specs=[
            pl.BlockSpec((512, IN), lambda i: (i, 0)),
            pl.BlockSpec((IN, OUT), lambda i: (0, 0)),
            pl.BlockSpec((1, OUT), lambda i: (0, 0)),
            pl.BlockSpec((OUT, 16), lambda i: (0, 0)),
            pl.BlockSpec((OUT, 16), lambda i: (0, 0)),
        ],
        out_specs=[
            pl.BlockSpec((512, OUT), lambda i: (i, 0)),
            pl.BlockSpec((512, 16), lambda i: (i, 0)),
            pl.BlockSpec((512, 16), lambda i: (i, 0)),
        ],
        out_shape=[
            jax.ShapeDtypeStruct((NP, OUT), jnp.float32),
            jax.ShapeDtypeStruct((NP, 16), jnp.float32),
            jax.ShapeDtypeStruct((NP, 16), jnp.float32),
        ],
    )(x_pad, wt, b2, ms, md)


def _k2_body(cidxp, as_hbm, ad_hbm,
             ex_hbm, den_hbm,
             cidx, s_ch, d_ch, ex_ch, zbuf, den_sh,
             sem0, sem1, sem2, sem3, sem4, sem5):
    cid = lax.axis_index("c")
    sid = lax.axis_index("s")
    wid = cid * NS + sid

    def _z(i, c):
        zbuf[i, :] = jnp.zeros((16,), jnp.float32)
        return c
    lax.fori_loop(0, 40, _z, 0)

    def _z2(t, c):
        pltpu.sync_copy(zbuf, den_sh.at[pl.ds(sid * ROWS_T + t * 40, 40)])
        return c
    lax.fori_loop(0, ROWS_T // 40, _z2, 0)
    plsc.subcore_barrier()

    gsem = (sem0, sem1)
    ssem = (sem2, sem3)
    esem = (sem4, sem5)

    def _fetch(j, p):
        row = wid * CPW + j
        pltpu.sync_copy(cidxp.at[pl.ds(row, 1)], cidx.at[pl.ds(p, 1)])
        pltpu.async_copy(as_hbm.at[cidx.at[p, 0]], s_ch.at[p], gsem[p])
        pltpu.async_copy(ad_hbm.at[cidx.at[p, 1]], d_ch.at[p], gsem[p])

    def _wait_fetch(p):
        pltpu.make_async_copy(as_hbm.at[cidx.at[p, 0]], s_ch.at[p],
                              gsem[p]).wait()
        pltpu.make_async_copy(ad_hbm.at[cidx.at[p, 1]], d_ch.at[p],
                              gsem[p]).wait()

    def _compute_scatter(j, p):
        row = wid * CPW + j

        def _edge(e, c2):
            v = s_ch[p, e, :] + d_ch[p, e, :]
            v = jnp.where(v > 0, v, v * jnp.float32(0.2))
            ex_ch[p, e, :] = jnp.exp(v)
            return c2
        lax.fori_loop(0, CH, _edge, 0)
        pltpu.async_copy(ex_ch.at[p], den_sh.at[cidx.at[p, 1]], ssem[p],
                         add=True)
        pltpu.async_copy(ex_ch.at[p], ex_hbm.at[pl.ds(row * CH, CH)],
                         esem[p])

    def _wait_scatter(p):
        pltpu.make_async_copy(ex_ch.at[p], den_sh.at[cidx.at[p, 1]],
                              ssem[p]).wait()
        pltpu.make_async_copy(ex_ch.at[p], ex_hbm.at[pl.ds(0, CH)],
                              esem[p]).wait()

    _fetch(0, 0)
    _fetch(1, 1)
    _wait_fetch(0)
    _compute_scatter(0, 0)

    def _step(j, p):
        q = 1 - p
        _wait_fetch(p)
        _wait_scatter(q)
        _fetch(j + 1, q)
        _compute_scatter(j, p)

    def _steady(jj, c):
        j = 2 * jj + 1
        _step(j, 1)
        _step(j + 1, 0)
        return c
    lax.fori_loop(0, (CPW - 2) // 2, _steady, 0)

    p = (CPW - 1) % 2
    _wait_fetch(p)
    _wait_scatter(1 - p)
    _compute_scatter(CPW - 1, p)
    _wait_scatter(p)

    plsc.subcore_barrier()
    pltpu.sync_copy(den_sh.at[pl.ds(sid * ROWS_T, ROWS_T)],
                    den_hbm.at[cid, pl.ds(sid * ROWS_T, ROWS_T)])


def _k2(cidxp, as_t, ad_t):
    mesh = plsc.VectorSubcoreMesh(core_axis_name="c", subcore_axis_name="s")
    f = pl.kernel(
        _k2_body,
        compiler_params=pltpu.CompilerParams(use_tc_tiling_on_sc=False),
        out_type=[
            jax.ShapeDtypeStruct((EP, 16), jnp.float32),
            jax.ShapeDtypeStruct((NC, NP, 16), jnp.float32),
        ],
        mesh=mesh,
        scratch_types=[
            pltpu.VMEM((2, 2, CH), jnp.int32),
            pltpu.VMEM((2, CH, 16), jnp.float32),
            pltpu.VMEM((2, CH, 16), jnp.float32),
            pltpu.VMEM((2, CH, 16), jnp.float32),
            pltpu.VMEM((40, 16), jnp.float32),
            pltpu.VMEM_SHARED((NP, 16), jnp.float32),
            pltpu.SemaphoreType.DMA,
            pltpu.SemaphoreType.DMA,
            pltpu.SemaphoreType.DMA,
            pltpu.SemaphoreType.DMA,
            pltpu.SemaphoreType.DMA,
            pltpu.SemaphoreType.DMA,
        ],
    )
    return f(cidxp, as_t, ad_t)


def _k2b_body(den_ref, r_ref):
    r_ref[...] = 1.0 / (den_ref[0] + den_ref[1] + jnp.float32(1e-16))


def _k2b(den):
    return pl.pallas_call(
        _k2b_body,
        out_shape=jax.ShapeDtypeStruct((NP, 16), jnp.float32),
    )(den)


def _k3_body(cidxp, ex_hbm, r_hbm, h_hbm,
             agg_hbm,
             cidx, r_ch, ex_ch, hs, zbuf,
             agg_sh, gsem0, gsem1, ssem0, ssem1, isem0, isem1, isem2):
    cid = lax.axis_index("c")
    sid = lax.axis_index("s")
    wid = cid * NS + sid
    gsem = (gsem0, gsem1)
    ssem = (ssem0, ssem1)
    isem = (isem0, isem1, isem2)

    def _z(i, c):
        r = i // 8
        cc = i % 8
        zbuf[r, pl.ds(cc * 16, 16)] = jnp.zeros((16,), jnp.float32)
        return c
    lax.fori_loop(0, 8 * 8, _z, 0)

    def _z2(t, c):
        pltpu.sync_copy(zbuf, agg_sh.at[pl.ds(sid * ROWS_T + t * 8, 8)])
        return c
    lax.fori_loop(0, ROWS_T // 8, _z2, 0)
    plsc.subcore_barrier()

    def _pipeline(base, cpw):
        def _issue_idx(j, r):
            row = base + jnp.minimum(j, cpw - 1)
            pltpu.async_copy(cidxp.at[pl.ds(row, 1)], cidx.at[pl.ds(r, 1)],
                             isem[r])

        def _wait_idx(r):
            pltpu.make_async_copy(cidxp.at[pl.ds(0, 1)],
                                  cidx.at[pl.ds(r, 1)], isem[r]).wait()

        def _gathers(j, p, r):
            row = base + j
            pltpu.async_copy(r_hbm.at[cidx.at[r, 1]], r_ch.at[p], gsem[p])
            pltpu.async_copy(h_hbm.at[cidx.at[r, 0]], hs.at[p], gsem[p])
            pltpu.async_copy(ex_hbm.at[pl.ds(row * CH, CH)], ex_ch.at[p],
                             gsem[p])

        def _wait_fetch(p, r):
            pltpu.make_async_copy(r_hbm.at[cidx.at[r, 1]], r_ch.at[p],
                                  gsem[p]).wait()
            pltpu.make_async_copy(h_hbm.at[cidx.at[r, 0]], hs.at[p],
                                  gsem[p]).wait()
            pltpu.make_async_copy(ex_hbm.at[pl.ds(0, CH)], ex_ch.at[p],
                                  gsem[p]).wait()

        def _compute_scatter(p, r):
            def _e(e, c2):
                wrow = ex_ch[p, e, :] * r_ch[p, e, :]
                for hh in range(H):
                    hs[p, e, pl.ds(hh * D, D)] = (hs[p, e, pl.ds(hh * D, D)]
                                                  * wrow[hh])
                return c2
            lax.fori_loop(0, CH, _e, 0)
            pltpu.async_copy(hs.at[p], agg_sh.at[cidx.at[r, 1]], ssem[p],
                             add=True)

        def _wait_scatter(p, r):
            pltpu.make_async_copy(hs.at[p], agg_sh.at[cidx.at[r, 1]],
                                  ssem[p]).wait()

        pltpu.sync_copy(cidxp.at[pl.ds(base, 1)], cidx.at[pl.ds(0, 1)])
        pltpu.sync_copy(cidxp.at[pl.ds(base + 1, 1)], cidx.at[pl.ds(1, 1)])
        _issue_idx(2, 2)
        _gathers(0, 0, 0)
        _gathers(1, 1, 1)
        _wait_fetch(0, 0)
        _compute_scatter(0, 0)

        def _stepx(j, p, rj, r1, r2):
            q = 1 - p
            _wait_fetch(p, rj)
            _wait_scatter(q, r2)
            _wait_idx(r1)
            _gathers(j + 1, q, r1)
            _issue_idx(j + 2, r2)
            _compute_scatter(p, rj)

        def _steady(jj, c):
            for m in range(6):
                j = 6 * jj + 1 + m
                _stepx(j, (1 + m) % 2, (1 + m) % 3, (2 + m) % 3, m % 3)
            return c
        lax.fori_loop(0, (cpw - 2) // 6, _steady, 0)

        p = (cpw - 1) % 2
        rj = (cpw - 1) % 3
        _wait_fetch(p, rj)
        _wait_scatter(1 - p, (cpw - 2) % 3)
        _wait_idx(cpw % 3)
        _compute_scatter(p, rj)
        _wait_scatter(p, rj)

    @pl.when(cid == 0)
    def _core0():
        _pipeline(sid * CPW0, CPW0)

    @pl.when(cid == 1)
    def _core1():
        _pipeline(NS * CPW0 + sid * CPW1, CPW1)

    plsc.subcore_barrier()
    pltpu.sync_copy(agg_sh.at[pl.ds(sid * ROWS_T, ROWS_T)],
                    agg_hbm.at[cid, pl.ds(sid * ROWS_T, ROWS_T)])


def _k3(cidxp, ex, r, h):
    mesh = plsc.VectorSubcoreMesh(core_axis_name="c", subcore_axis_name="s")
    f = pl.kernel(
        _k3_body,
        compiler_params=pltpu.CompilerParams(use_tc_tiling_on_sc=False),
        out_type=jax.ShapeDtypeStruct((NC, NP, OUT), jnp.float32),
        mesh=mesh,
        scratch_types=[
            pltpu.VMEM((3, 2, CH), jnp.int32),
            pltpu.VMEM((2, CH, 16), jnp.float32),
            pltpu.VMEM((2, CH, 16), jnp.float32),
            pltpu.VMEM((2, CH, OUT), jnp.float32),
            pltpu.VMEM((8, 128), jnp.float32),
            pltpu.VMEM_SHARED((NP, OUT), jnp.float32),
            pltpu.SemaphoreType.DMA,
            pltpu.SemaphoreType.DMA,
            pltpu.SemaphoreType.DMA,
            pltpu.SemaphoreType.DMA,
            pltpu.SemaphoreType.DMA,
            pltpu.SemaphoreType.DMA,
            pltpu.SemaphoreType.DMA,
        ],
    )
    return f(cidxp, ex, r, h)


def _k4_body(agg_ref, g_ref, b_ref, o_ref):
    out = jax.nn.relu(agg_ref[0, :N, :] + agg_ref[1, :N, :])
    mean = jnp.mean(out, axis=0, keepdims=True)
    var = jnp.mean((out - mean) ** 2, axis=0, keepdims=True)
    o_ref[...] = (out - mean) / jnp.sqrt(var + 1e-5) * g_ref[...] + b_ref[...]


def _k4(agg, gamma, beta):
    return pl.pallas_call(
        _k4_body,
        out_shape=jax.ShapeDtypeStruct((N, OUT), jnp.float32),
    )(agg, gamma.reshape(1, OUT), beta.reshape(1, OUT))


def kernel(x, edge_index, W_proj, b_proj, att_src, att_dst, k_W, k_b, q,
           bn_gamma, bn_beta):
    mask = jnp.concatenate([jnp.eye(H, dtype=jnp.float32),
                            jnp.zeros((H, H), jnp.float32)], axis=1)
    ms = (att_src.reshape(H, D)[:, :, None] * mask[:, None, :]).reshape(OUT, 16)
    md = (att_dst.reshape(H, D)[:, :, None] * mask[:, None, :]).reshape(OUT, 16)

    x_pad = jnp.zeros((NP, IN), jnp.float32).at[:N, :].set(x)
    h, as_t, ad_t = _k1(x_pad, W_proj.T, b_proj.reshape(1, OUT), ms, md)

    src = edge_index[0]
    dst = edge_index[1]
    pad_idx = jnp.full((EP - E,), N, jnp.int32)
    srcp = jnp.concatenate([src, pad_idx]).reshape(NW * CPW, CH)
    dstp = jnp.concatenate([dst, pad_idx]).reshape(NW * CPW, CH)
    cidxp = jnp.stack([srcp, dstp], axis=1)

    ex, den = _k2(cidxp, as_t, ad_t)
    r = _k2b(den)
    agg = _k3(cidxp, ex, r, h)
    return _k4(agg, bn_gamma, bn_beta)
